# baseline (speedup 1.0000x reference)
import functools

import jax
import jax.numpy as jnp
from jax import lax
from jax.experimental import pallas as pl
from jax.experimental.pallas import tpu as pltpu
from jax.experimental.pallas import tpu_sc as plsc

N = 10000
E = 320000
D = 128
NC = 2
NS = 16
NW = NC * NS
EPW = E // NW
C = 80
NCHUNK = EPW // C
RPT = 624
RREM = N - NS * RPT

_mesh = plsc.VectorSubcoreMesh(
    core_axis_name="c", subcore_axis_name="s", num_cores=NC, num_subcores=NS)


NBUF = 3
NIB = 6


def _zero_acc(rows_v, acc_sh, s):
    def zrow(i, carry):
        for j in range(D // 16):
            rows_v[NBUF - 1, i, pl.ds(j * 16, 16)] = jnp.zeros((16,),
                                                              jnp.float32)
        return carry

    lax.fori_loop(0, C, zrow, 0)
    for r in range(RPT // C):
        pltpu.sync_copy(rows_v.at[NBUF - 1],
                        acc_sh.at[pl.ds(s * RPT + r * C, C)])
    pltpu.sync_copy(rows_v.at[NBUF - 1, pl.ds(0, RPT % C)],
                    acc_sh.at[pl.ds(s * RPT + (RPT // C) * C, RPT % C)])

    @pl.when(s == 0)
    def _():
        pltpu.sync_copy(rows_v.at[NBUF - 1, pl.ds(0, RREM)],
                        acc_sh.at[pl.ds(NS * RPT, RREM)])


def _write_acc(acc_sh, acc_out, c, s):
    pltpu.sync_copy(acc_sh.at[pl.ds(s * RPT, RPT)],
                    acc_out.at[c, pl.ds(s * RPT, RPT)])

    @pl.when(s == 0)
    def _():
        pltpu.sync_copy(acc_sh.at[pl.ds(NS * RPT, RREM)],
                        acc_out.at[c, pl.ds(NS * RPT, RREM)])


def _agg_pipeline(y_hbm, eidx_hbm, idx_v, rows_v, acc_sh,
                  gsems, isems, ssems, w, deg_start, deg_wait, init_fn):
    def stage_idx(k, bi):
        pltpu.async_copy(eidx_hbm.at[w, k], idx_v.at[bi], isems[bi])

    def wait_idx(k, bi):
        pltpu.make_async_copy(eidx_hbm.at[w, k], idx_v.at[bi],
                              isems[bi]).wait()

    def start_gather(k, bi, br):
        pltpu.async_copy(y_hbm.at[idx_v.at[bi, 0]], rows_v.at[br], gsems[br])

    def wait_gather(bi, br):
        pltpu.make_async_copy(y_hbm.at[idx_v.at[bi, 0]], rows_v.at[br],
                              gsems[br]).wait()

    def start_scatter(bi, br):
        pltpu.async_copy(rows_v.at[br], acc_sh.at[idx_v.at[bi, 1]], ssems[br],
                         add=True)

    def wait_scatter(bi, br):
        pltpu.make_async_copy(rows_v.at[br], acc_sh.at[idx_v.at[bi, 1]],
                              ssems[br]).wait()

    for j in range(4):
        stage_idx(j, j)
    for j in range(2):
        wait_idx(j, j)
        start_gather(j, j, j)
    init_fn()

    def outer(g, carry):
        for j in range(NIB):
            k = g * NIB + j
            br = j % NBUF
            b2r = (j + 2) % NBUF
            bi = j
            bi1 = (j + 5) % NIB
            bi2 = (j + 2) % NIB
            bi4 = (j + 4) % NIB

            @pl.when(k < NCHUNK)
            def _():
                wait_gather(bi, br)
                start_scatter(bi, br)
                deg_start(bi, br)

            @pl.when(jnp.logical_and(k >= 1, k + 2 < NCHUNK))
            def _():
                wait_scatter(bi1, b2r)
                deg_wait(bi1, b2r)

            @pl.when(k + 2 < NCHUNK)
            def _():
                wait_idx(k + 2, bi2)
                start_gather(k + 2, bi2, b2r)

            @pl.when(k + 4 < NCHUNK)
            def _():
                stage_idx(k + 4, bi4)

        return carry

    lax.fori_loop(0, -(-NCHUNK // NIB), outer, 0)
    for k in range(NCHUNK - NBUF, NCHUNK):
        wait_scatter(k % NIB, k % NBUF)
        deg_wait(k % NIB, k % NBUF)


_SC_SEMS = [pltpu.SemaphoreType.DMA] * 12


def _sc_agg_deg_body(y_hbm, eidx_hbm, zdeg_hbm,
                     acc_out, deg_out,
                     idx_v, rows_v, ones_v, acc_sh, deg_sh, *sems):
    c = lax.axis_index("c")
    s = lax.axis_index("s")
    w = c * NS + s
    gsems, isems, ssems = sems[0:3], sems[3:9], sems[9:12]
    for j in range(C // 16):
        ones_v[pl.ds(j * 16, 16)] = jnp.ones((16,), jnp.float32)

    def init_fn():
        _zero_acc(rows_v, acc_sh, s)

        @pl.when(s == 0)
        def _():
            pltpu.sync_copy(zdeg_hbm, deg_sh)

        plsc.subcore_barrier()

    def deg_start(bi, br):
        pltpu.async_copy(ones_v, deg_sh.at[idx_v.at[bi, 1]], ssems[br],
                         add=True)

    def deg_wait(bi, br):
        pltpu.make_async_copy(ones_v, deg_sh.at[idx_v.at[bi, 1]],
                              ssems[br]).wait()

    _agg_pipeline(y_hbm, eidx_hbm, idx_v, rows_v, acc_sh,
                  gsems, isems, ssems, w, deg_start, deg_wait, init_fn)
    plsc.subcore_barrier()
    _write_acc(acc_sh, acc_out, c, s)

    @pl.when(s == 0)
    def _():
        pltpu.sync_copy(deg_sh, deg_out.at[c])


_sc_agg_deg = functools.partial(
    pl.kernel,
    out_type=(jax.ShapeDtypeStruct((NC, N, D), jnp.float32),
              jax.ShapeDtypeStruct((NC, N), jnp.float32)),
    mesh=_mesh,
    scratch_types=[
        pltpu.VMEM((NIB, 2, C), jnp.int32),
        pltpu.VMEM((NBUF, C, D), jnp.float32),
        pltpu.VMEM((C,), jnp.float32),
        pltpu.VMEM_SHARED((N, D), jnp.float32),
        pltpu.VMEM_SHARED((N,), jnp.float32),
    ] + _SC_SEMS,
)(_sc_agg_deg_body)


def _sc_agg_body(y_hbm, eidx_hbm,
                 acc_out,
                 idx_v, rows_v, acc_sh, *sems):
    c = lax.axis_index("c")
    s = lax.axis_index("s")
    w = c * NS + s
    gsems, isems, ssems = sems[0:3], sems[3:9], sems[9:12]

    def init_fn():
        _zero_acc(rows_v, acc_sh, s)
        plsc.subcore_barrier()

    _agg_pipeline(y_hbm, eidx_hbm, idx_v, rows_v, acc_sh,
                  gsems, isems, ssems, w,
                  lambda bi, br: None, lambda bi, br: None, init_fn)
    plsc.subcore_barrier()
    _write_acc(acc_sh, acc_out, c, s)


_sc_agg = functools.partial(
    pl.kernel,
    out_type=jax.ShapeDtypeStruct((NC, N, D), jnp.float32),
    mesh=_mesh,
    scratch_types=[
        pltpu.VMEM((NIB, 2, C), jnp.int32),
        pltpu.VMEM((NBUF, C, D), jnp.float32),
        pltpu.VMEM_SHARED((N, D), jnp.float32),
    ] + _SC_SEMS,
)(_sc_agg_body)


def _dot(a, b):
    return jnp.dot(a, b, preferred_element_type=jnp.float32)


def _tc_in_body(h_ref, w0_ref, b0_ref, w1_ref, b1_ref, x_ref):
    t = jnp.tanh(_dot(h_ref[...], w0_ref[...]) + b0_ref[...])
    x_ref[...] = _dot(t, w1_ref[...]) + b1_ref[...]


_tc_in = pl.pallas_call(
    _tc_in_body,
    out_shape=jax.ShapeDtypeStruct((N, D), jnp.float32),
)


def _sage_post(x_ref, acc_ref, degp_ref, ws_ref, wn_ref, bb_ref):
    deg = degp_ref[0, :] + degp_ref[1, :]
    inv = 1.0 / jnp.maximum(deg, 1.0)
    hn = (acc_ref[0] + acc_ref[1]) * inv[:, None]
    return (_dot(x_ref[...], ws_ref[...]) + _dot(hn, wn_ref[...])
            + bb_ref[...])


def _tc_mid_body(x_ref, acc_ref, degp_ref, ws_ref, wn_ref, bb_ref, out_ref):
    out_ref[...] = jax.nn.silu(
        _sage_post(x_ref, acc_ref, degp_ref, ws_ref, wn_ref, bb_ref))


_tc_mid = pl.pallas_call(
    _tc_mid_body,
    out_shape=jax.ShapeDtypeStruct((N, D), jnp.float32),
)


def _tc_out_body(x_ref, acc_ref, degp_ref, ws_ref, wn_ref, bb_ref, out_ref):
    out_ref[...] = jnp.tanh(
        _sage_post(x_ref, acc_ref, degp_ref, ws_ref, wn_ref, bb_ref))


_tc_out = pl.pallas_call(
    _tc_out_body,
    out_shape=jax.ShapeDtypeStruct((N, D), jnp.float32),
)


def kernel(h, edge_index, W0, b0, W1, b1, Ws0, Wn0, bb0, Ws1, Wn1, bb1,
           Ws2, Wn2, bb2):
    eidx = jnp.stack(
        [edge_index[0].astype(jnp.int32).reshape(NW, NCHUNK, C),
         edge_index[1].astype(jnp.int32).reshape(NW, NCHUNK, C)], axis=2)
    b0r = b0.reshape(1, D)
    b1r = b1.reshape(1, D)
    bb0r = bb0.reshape(1, D)
    bb1r = bb1.reshape(1, D)
    bb2r = bb2.reshape(1, D)
    zdeg = jnp.zeros((N,), jnp.float32)

    x0 = _tc_in(h, W0, b0r, W1, b1r)
    acc0, degp = _sc_agg_deg(x0, eidx, zdeg)
    x1 = _tc_mid(x0, acc0, degp, Ws0, Wn0, bb0r)
    acc1 = _sc_agg(x1, eidx)
    x2 = _tc_mid(x1, acc1, degp, Ws1, Wn1, bb1r)
    acc2 = _sc_agg(x2, eidx)
    return _tc_out(x2, acc2, degp, Ws2, Wn2, bb2r)


# speedup vs baseline: 1.0332x; 1.0332x over previous
import functools

import jax
import jax.numpy as jnp
from jax import lax
from jax.experimental import pallas as pl
from jax.experimental.pallas import tpu as pltpu
from jax.experimental.pallas import tpu_sc as plsc

N = 10000
E = 320000
D = 128
NC = 2
NS = 16
NW = NC * NS
EPW = E // NW
C = 80
NCHUNK = EPW // C
RPT = 624
RREM = N - NS * RPT

_mesh = plsc.VectorSubcoreMesh(
    core_axis_name="c", subcore_axis_name="s", num_cores=NC, num_subcores=NS)


NBUF = 3
NIB = 6


def _zero_acc(rows_v, acc_sh, s):
    def zrow(i, carry):
        for j in range(D // 16):
            rows_v[NBUF - 1, i, pl.ds(j * 16, 16)] = jnp.zeros((16,),
                                                              jnp.float32)
        return carry

    lax.fori_loop(0, C, zrow, 0)
    for r in range(RPT // C):
        pltpu.sync_copy(rows_v.at[NBUF - 1],
                        acc_sh.at[pl.ds(s * RPT + r * C, C)])
    pltpu.sync_copy(rows_v.at[NBUF - 1, pl.ds(0, RPT % C)],
                    acc_sh.at[pl.ds(s * RPT + (RPT // C) * C, RPT % C)])

    @pl.when(s == 0)
    def _():
        pltpu.sync_copy(rows_v.at[NBUF - 1, pl.ds(0, RREM)],
                        acc_sh.at[pl.ds(NS * RPT, RREM)])


def _write_acc(acc_sh, acc_out, c, s):
    pltpu.sync_copy(acc_sh.at[pl.ds(s * RPT, RPT)],
                    acc_out.at[c, pl.ds(s * RPT, RPT)])

    @pl.when(s == 0)
    def _():
        pltpu.sync_copy(acc_sh.at[pl.ds(NS * RPT, RREM)],
                        acc_out.at[c, pl.ds(NS * RPT, RREM)])


def _agg_pipeline(y_hbm, src3_hbm, dst3_hbm, src_v, dst_v, rows_v, acc_sh,
                  gsems, isems, ssems, w, deg_start, deg_wait, init_fn):
    def stage_idx(k, bi):
        pltpu.async_copy(src3_hbm.at[w, k], src_v.at[bi], isems[bi])
        pltpu.async_copy(dst3_hbm.at[w, k], dst_v.at[bi], isems[bi])

    def wait_idx(k, bi):
        pltpu.make_async_copy(src3_hbm.at[w, k], src_v.at[bi],
                              isems[bi]).wait()
        pltpu.make_async_copy(dst3_hbm.at[w, k], dst_v.at[bi],
                              isems[bi]).wait()

    def start_gather(k, bi, br):
        pltpu.async_copy(y_hbm.at[src_v.at[bi]], rows_v.at[br], gsems[br])

    def wait_gather(bi, br):
        pltpu.make_async_copy(y_hbm.at[src_v.at[bi]], rows_v.at[br],
                              gsems[br]).wait()

    def start_scatter(bi, br):
        pltpu.async_copy(rows_v.at[br], acc_sh.at[dst_v.at[bi]], ssems[br],
                         add=True)

    def wait_scatter(bi, br):
        pltpu.make_async_copy(rows_v.at[br], acc_sh.at[dst_v.at[bi]],
                              ssems[br]).wait()

    for j in range(4):
        stage_idx(j, j)
    for j in range(2):
        wait_idx(j, j)
        start_gather(j, j, j)
    init_fn()

    def outer(g, carry):
        for j in range(NIB):
            k = g * NIB + j
            br = j % NBUF
            b2r = (j + 2) % NBUF
            bi = j
            bi1 = (j + 5) % NIB
            bi2 = (j + 2) % NIB
            bi4 = (j + 4) % NIB

            @pl.when(k < NCHUNK)
            def _():
                wait_gather(bi, br)
                start_scatter(bi, br)
                deg_start(bi, br)

            @pl.when(jnp.logical_and(k >= 1, k + 2 < NCHUNK))
            def _():
                wait_scatter(bi1, b2r)
                deg_wait(bi1, b2r)

            @pl.when(k + 2 < NCHUNK)
            def _():
                wait_idx(k + 2, bi2)
                start_gather(k + 2, bi2, b2r)

            @pl.when(k + 4 < NCHUNK)
            def _():
                stage_idx(k + 4, bi4)

        return carry

    lax.fori_loop(0, -(-NCHUNK // NIB), outer, 0)
    for k in range(NCHUNK - NBUF, NCHUNK):
        wait_scatter(k % NIB, k % NBUF)
        deg_wait(k % NIB, k % NBUF)


_SC_SEMS = [pltpu.SemaphoreType.DMA] * 12


def _sc_agg_deg_body(y_hbm, src3_hbm, dst3_hbm, zdeg_hbm,
                     acc_out, deg_out,
                     src_v, dst_v, rows_v, ones_v, acc_sh, deg_sh, *sems):
    c = lax.axis_index("c")
    s = lax.axis_index("s")
    w = c * NS + s
    gsems, isems, ssems = sems[0:3], sems[3:9], sems[9:12]
    for j in range(C // 16):
        ones_v[pl.ds(j * 16, 16)] = jnp.ones((16,), jnp.float32)

    def init_fn():
        _zero_acc(rows_v, acc_sh, s)

        @pl.when(s == 0)
        def _():
            pltpu.sync_copy(zdeg_hbm, deg_sh)

        plsc.subcore_barrier()

    def deg_start(bi, br):
        pltpu.async_copy(ones_v, deg_sh.at[dst_v.at[bi]], ssems[br], add=True)

    def deg_wait(bi, br):
        pltpu.make_async_copy(ones_v, deg_sh.at[dst_v.at[bi]],
                              ssems[br]).wait()

    _agg_pipeline(y_hbm, src3_hbm, dst3_hbm, src_v, dst_v, rows_v, acc_sh,
                  gsems, isems, ssems, w, deg_start, deg_wait, init_fn)
    plsc.subcore_barrier()
    _write_acc(acc_sh, acc_out, c, s)

    @pl.when(s == 0)
    def _():
        pltpu.sync_copy(deg_sh, deg_out.at[c])


_sc_agg_deg = functools.partial(
    pl.kernel,
    out_type=(jax.ShapeDtypeStruct((NC, N, D), jnp.float32),
              jax.ShapeDtypeStruct((NC, N), jnp.float32)),
    mesh=_mesh,
    scratch_types=[
        pltpu.VMEM((NIB, C), jnp.int32),
        pltpu.VMEM((NIB, C), jnp.int32),
        pltpu.VMEM((NBUF, C, D), jnp.float32),
        pltpu.VMEM((C,), jnp.float32),
        pltpu.VMEM_SHARED((N, D), jnp.float32),
        pltpu.VMEM_SHARED((N,), jnp.float32),
    ] + _SC_SEMS,
)(_sc_agg_deg_body)


def _sc_agg_body(y_hbm, src3_hbm, dst3_hbm,
                 acc_out,
                 src_v, dst_v, rows_v, acc_sh, *sems):
    c = lax.axis_index("c")
    s = lax.axis_index("s")
    w = c * NS + s
    gsems, isems, ssems = sems[0:3], sems[3:9], sems[9:12]

    def init_fn():
        _zero_acc(rows_v, acc_sh, s)
        plsc.subcore_barrier()

    _agg_pipeline(y_hbm, src3_hbm, dst3_hbm, src_v, dst_v, rows_v, acc_sh,
                  gsems, isems, ssems, w,
                  lambda bi, br: None, lambda bi, br: None, init_fn)
    plsc.subcore_barrier()
    _write_acc(acc_sh, acc_out, c, s)


_sc_agg = functools.partial(
    pl.kernel,
    out_type=jax.ShapeDtypeStruct((NC, N, D), jnp.float32),
    mesh=_mesh,
    scratch_types=[
        pltpu.VMEM((NIB, C), jnp.int32),
        pltpu.VMEM((NIB, C), jnp.int32),
        pltpu.VMEM((NBUF, C, D), jnp.float32),
        pltpu.VMEM_SHARED((N, D), jnp.float32),
    ] + _SC_SEMS,
)(_sc_agg_body)


def _dot(a, b):
    return jnp.dot(a, b, preferred_element_type=jnp.float32)


def _tc_in_body(h_ref, w0_ref, b0_ref, w1_ref, b1_ref, x_ref):
    t = jnp.tanh(_dot(h_ref[...], w0_ref[...]) + b0_ref[...])
    x_ref[...] = _dot(t, w1_ref[...]) + b1_ref[...]


_tc_in = pl.pallas_call(
    _tc_in_body,
    out_shape=jax.ShapeDtypeStruct((N, D), jnp.float32),
)


def _sage_post(x_ref, acc_ref, degp_ref, ws_ref, wn_ref, bb_ref):
    deg = degp_ref[0, :] + degp_ref[1, :]
    inv = 1.0 / jnp.maximum(deg, 1.0)
    hn = (acc_ref[0] + acc_ref[1]) * inv[:, None]
    return (_dot(x_ref[...], ws_ref[...]) + _dot(hn, wn_ref[...])
            + bb_ref[...])


def _tc_mid_body(x_ref, acc_ref, degp_ref, ws_ref, wn_ref, bb_ref, out_ref):
    out_ref[...] = jax.nn.silu(
        _sage_post(x_ref, acc_ref, degp_ref, ws_ref, wn_ref, bb_ref))


_tc_mid = pl.pallas_call(
    _tc_mid_body,
    out_shape=jax.ShapeDtypeStruct((N, D), jnp.float32),
)


def _tc_out_body(x_ref, acc_ref, degp_ref, ws_ref, wn_ref, bb_ref, out_ref):
    out_ref[...] = jnp.tanh(
        _sage_post(x_ref, acc_ref, degp_ref, ws_ref, wn_ref, bb_ref))


_tc_out = pl.pallas_call(
    _tc_out_body,
    out_shape=jax.ShapeDtypeStruct((N, D), jnp.float32),
)


def kernel(h, edge_index, W0, b0, W1, b1, Ws0, Wn0, bb0, Ws1, Wn1, bb1,
           Ws2, Wn2, bb2):
    src = edge_index[0].astype(jnp.int32).reshape(NW, NCHUNK, C)
    dst = edge_index[1].astype(jnp.int32).reshape(NW, NCHUNK, C)
    b0r = b0.reshape(1, D)
    b1r = b1.reshape(1, D)
    bb0r = bb0.reshape(1, D)
    bb1r = bb1.reshape(1, D)
    bb2r = bb2.reshape(1, D)
    zdeg = jnp.zeros((N,), jnp.float32)

    x0 = _tc_in(h, W0, b0r, W1, b1r)
    acc0, degp = _sc_agg_deg(x0, src, dst, zdeg)
    x1 = _tc_mid(x0, acc0, degp, Ws0, Wn0, bb0r)
    acc1 = _sc_agg(x1, src, dst)
    x2 = _tc_mid(x1, acc1, degp, Ws1, Wn1, bb1r)
    acc2 = _sc_agg(x2, src, dst)
    return _tc_out(x2, acc2, degp, Ws2, Wn2, bb2r)
